# baseline (speedup 1.0000x reference)
import jax
import jax.numpy as jnp
from jax.experimental import pallas as pl
from jax.experimental.pallas import tpu as pltpu

_N = 10000
_R, _C = 8, 1250

_K0 = 0
_K1 = 42
_K2 = _K0 ^ _K1 ^ 0x1BD11BDA
_KS = (_K0, _K1, _K2)
_ROTS = ((13, 15, 26, 6), (17, 29, 16, 24))
_DROP_THRESH = 419430


def _keep_2d():
    r = jax.lax.broadcasted_iota(jnp.uint32, (_R, _C), 0)
    c = jax.lax.broadcasted_iota(jnp.uint32, (_R, _C), 1)
    p = r * jnp.uint32(_C) + c
    x0 = jnp.full((_R, _C), jnp.uint32(_K0))
    x1 = p + jnp.uint32(_K1)
    for i in range(5):
        for d in _ROTS[i % 2]:
            x0 = x0 + x1
            x1 = (x1 << jnp.uint32(d)) | (x1 >> jnp.uint32(32 - d))
            x1 = x1 ^ x0
        x0 = x0 + jnp.uint32(_KS[(i + 1) % 3])
        x1 = x1 + jnp.uint32(_KS[(i + 2) % 3] + i + 1)
    bits = x0 ^ x1
    return (bits >> jnp.uint32(9)) > jnp.uint32(_DROP_THRESH)


_XCHUNKS = 10
_ECHUNKS = 10


def _body(x_in, y_in, tr_in, te_in, e_in,
          x_out, e_out, y_out, tr_out, te_out,
          sem_big):
    copies = []
    xr = _N // _XCHUNKS
    for i in range(_XCHUNKS):
        sl = pl.ds(i * xr, xr)
        copies.append(pltpu.make_async_copy(
            x_in.at[sl, :], x_out.at[sl, :], sem_big.at[i]))
    ec = e_in.shape[1] // _ECHUNKS
    for i in range(_ECHUNKS):
        sl = pl.ds(i * ec, ec)
        copies.append(pltpu.make_async_copy(
            e_in.at[:, sl], e_out.at[:, sl], sem_big.at[_XCHUNKS + i]))
    copies.append(pltpu.make_async_copy(
        y_in, y_out, sem_big.at[_XCHUNKS + _ECHUNKS]))
    for c in copies:
        c.start()
    keep = _keep_2d()
    for r in range(_R):
        row = keep[r, :]
        sl = pl.ds(r * _C, _C)
        tr_out[sl] = jnp.logical_and(tr_in[sl], row)
        te_out[sl] = jnp.logical_and(te_in[sl], row)
    for c in copies:
        c.wait()


def kernel(x, y, train_mask, test_mask, edge_index):
    any_spec = pl.BlockSpec(memory_space=pl.ANY)
    vmem_spec = pl.BlockSpec(memory_space=pltpu.VMEM)
    x_o, e_o, y_o, tr_o, te_o = pl.pallas_call(
        _body,
        in_specs=[any_spec, any_spec, vmem_spec, vmem_spec, any_spec],
        out_specs=[any_spec, any_spec, any_spec, vmem_spec, vmem_spec],
        out_shape=(
            jax.ShapeDtypeStruct(x.shape, x.dtype),
            jax.ShapeDtypeStruct(edge_index.shape, edge_index.dtype),
            jax.ShapeDtypeStruct(y.shape, y.dtype),
            jax.ShapeDtypeStruct((_N,), jnp.bool_),
            jax.ShapeDtypeStruct((_N,), jnp.bool_),
        ),
        scratch_shapes=[
            pltpu.SemaphoreType.DMA((_XCHUNKS + _ECHUNKS + 1,)),
        ],
    )(x, y, train_mask, test_mask, edge_index)
    return (x_o, e_o, y_o, tr_o, te_o)


# speedup vs baseline: 15.9343x; 15.9343x over previous
import jax
import jax.numpy as jnp
from jax.experimental import pallas as pl
from jax.experimental.pallas import tpu as pltpu

_N = 10000
_R, _C = 8, 1250

_K0 = 0
_K1 = 42
_K2 = _K0 ^ _K1 ^ 0x1BD11BDA
_KS = (_K0, _K1, _K2)
_ROTS = ((13, 15, 26, 6), (17, 29, 16, 24))
_DROP_THRESH = 419430

_STEPS = 10
_XROWS = _N // _STEPS
_ECOLS = 320000 // _STEPS


def _keep_2d():
    r = jax.lax.broadcasted_iota(jnp.uint32, (_R, _C), 0)
    c = jax.lax.broadcasted_iota(jnp.uint32, (_R, _C), 1)
    p = r * jnp.uint32(_C) + c
    x0 = jnp.full((_R, _C), jnp.uint32(_K0))
    x1 = p + jnp.uint32(_K1)
    for i in range(5):
        for d in _ROTS[i % 2]:
            x0 = x0 + x1
            x1 = (x1 << jnp.uint32(d)) | (x1 >> jnp.uint32(32 - d))
            x1 = x1 ^ x0
        x0 = x0 + jnp.uint32(_KS[(i + 1) % 3])
        x1 = x1 + jnp.uint32(_KS[(i + 2) % 3] + i + 1)
    bits = x0 ^ x1
    return (bits >> jnp.uint32(9)) > jnp.uint32(_DROP_THRESH)


def _body(x_in, y_in, tr_in, te_in, e_in,
          x_out, e_out, y_out, tr_out, te_out):
    x_out[...] = x_in[...]
    e_out[...] = e_in[...]

    @pl.when(pl.program_id(0) == 0)
    def _once():
        y_out[...] = y_in[...]
        keep = _keep_2d()
        for r in range(_R):
            row = keep[r, :]
            sl = pl.ds(r * _C, _C)
            tr_out[sl] = jnp.logical_and(tr_in[sl], row)
            te_out[sl] = jnp.logical_and(te_in[sl], row)


def kernel(x, y, train_mask, test_mask, edge_index):
    once = lambda i: (0,)
    x_o, e_o, y_o, tr_o, te_o = pl.pallas_call(
        _body,
        grid=(_STEPS,),
        in_specs=[
            pl.BlockSpec((_XROWS, 128), lambda i: (i, 0)),
            pl.BlockSpec((_N,), once),
            pl.BlockSpec((_N,), once),
            pl.BlockSpec((_N,), once),
            pl.BlockSpec((2, _ECOLS), lambda i: (0, i)),
        ],
        out_specs=[
            pl.BlockSpec((_XROWS, 128), lambda i: (i, 0)),
            pl.BlockSpec((2, _ECOLS), lambda i: (0, i)),
            pl.BlockSpec((_N,), once),
            pl.BlockSpec((_N,), once),
            pl.BlockSpec((_N,), once),
        ],
        out_shape=(
            jax.ShapeDtypeStruct(x.shape, x.dtype),
            jax.ShapeDtypeStruct(edge_index.shape, edge_index.dtype),
            jax.ShapeDtypeStruct(y.shape, y.dtype),
            jax.ShapeDtypeStruct((_N,), jnp.bool_),
            jax.ShapeDtypeStruct((_N,), jnp.bool_),
        ),
    )(x, y, train_mask, test_mask, edge_index)
    return (x_o, e_o, y_o, tr_o, te_o)


# speedup vs baseline: 18.6002x; 1.1673x over previous
import jax
import jax.numpy as jnp
from jax.experimental import pallas as pl
from jax.experimental.pallas import tpu as pltpu

_N = 10000
_R, _C = 8, 1250

_K0 = 0
_K1 = 42
_K2 = _K0 ^ _K1 ^ 0x1BD11BDA
_KS = (_K0, _K1, _K2)
_ROTS = ((13, 15, 26, 6), (17, 29, 16, 24))
_DROP_THRESH = 419430

_STEPS = 5
_XROWS = _N // _STEPS
_ECOLS = 320000 // _STEPS


def _keep_2d():
    r = jax.lax.broadcasted_iota(jnp.uint32, (_R, _C), 0)
    c = jax.lax.broadcasted_iota(jnp.uint32, (_R, _C), 1)
    p = r * jnp.uint32(_C) + c
    x0 = jnp.full((_R, _C), jnp.uint32(_K0))
    x1 = p + jnp.uint32(_K1)
    for i in range(5):
        for d in _ROTS[i % 2]:
            x0 = x0 + x1
            x1 = (x1 << jnp.uint32(d)) | (x1 >> jnp.uint32(32 - d))
            x1 = x1 ^ x0
        x0 = x0 + jnp.uint32(_KS[(i + 1) % 3])
        x1 = x1 + jnp.uint32(_KS[(i + 2) % 3] + i + 1)
    bits = x0 ^ x1
    return (bits >> jnp.uint32(9)) > jnp.uint32(_DROP_THRESH)


def _body(x_in, y_in, tr_in, te_in, e_in,
          x_out, e_out, y_out, tr_out, te_out):
    x_out[...] = x_in[...]
    e_out[...] = e_in[...]

    @pl.when(pl.program_id(0) == 0)
    def _once():
        y_out[...] = y_in[...]
        keep = _keep_2d()
        for r in range(_R):
            row = keep[r, :]
            sl = pl.ds(r * _C, _C)
            tr_out[sl] = jnp.logical_and(tr_in[sl], row)
            te_out[sl] = jnp.logical_and(te_in[sl], row)


def kernel(x, y, train_mask, test_mask, edge_index):
    once = lambda i: (0,)
    x_o, e_o, y_o, tr_o, te_o = pl.pallas_call(
        _body,
        grid=(_STEPS,),
        in_specs=[
            pl.BlockSpec((_XROWS, 128), lambda i: (i, 0)),
            pl.BlockSpec((_N,), once),
            pl.BlockSpec((_N,), once),
            pl.BlockSpec((_N,), once),
            pl.BlockSpec((2, _ECOLS), lambda i: (0, i)),
        ],
        out_specs=[
            pl.BlockSpec((_XROWS, 128), lambda i: (i, 0)),
            pl.BlockSpec((2, _ECOLS), lambda i: (0, i)),
            pl.BlockSpec((_N,), once),
            pl.BlockSpec((_N,), once),
            pl.BlockSpec((_N,), once),
        ],
        out_shape=(
            jax.ShapeDtypeStruct(x.shape, x.dtype),
            jax.ShapeDtypeStruct(edge_index.shape, edge_index.dtype),
            jax.ShapeDtypeStruct(y.shape, y.dtype),
            jax.ShapeDtypeStruct((_N,), jnp.bool_),
            jax.ShapeDtypeStruct((_N,), jnp.bool_),
        ),
    )(x, y, train_mask, test_mask, edge_index)
    return (x_o, e_o, y_o, tr_o, te_o)


# speedup vs baseline: 21.4430x; 1.1528x over previous
import jax
import jax.numpy as jnp
from jax.experimental import pallas as pl
from jax.experimental.pallas import tpu as pltpu

_N = 10000
_R, _C = 8, 1250

_K0 = 0
_K1 = 42
_K2 = _K0 ^ _K1 ^ 0x1BD11BDA
_KS = (_K0, _K1, _K2)
_ROTS = ((13, 15, 26, 6), (17, 29, 16, 24))
_DROP_THRESH = 419430

_STEPS = 2
_XROWS = _N // _STEPS
_ECOLS = 320000 // _STEPS


def _keep_2d():
    r = jax.lax.broadcasted_iota(jnp.uint32, (_R, _C), 0)
    c = jax.lax.broadcasted_iota(jnp.uint32, (_R, _C), 1)
    p = r * jnp.uint32(_C) + c
    x0 = jnp.full((_R, _C), jnp.uint32(_K0))
    x1 = p + jnp.uint32(_K1)
    for i in range(5):
        for d in _ROTS[i % 2]:
            x0 = x0 + x1
            x1 = (x1 << jnp.uint32(d)) | (x1 >> jnp.uint32(32 - d))
            x1 = x1 ^ x0
        x0 = x0 + jnp.uint32(_KS[(i + 1) % 3])
        x1 = x1 + jnp.uint32(_KS[(i + 2) % 3] + i + 1)
    bits = x0 ^ x1
    return (bits >> jnp.uint32(9)) > jnp.uint32(_DROP_THRESH)


def _body(x_in, y_in, tr_in, te_in, e_in,
          x_out, e_out, y_out, tr_out, te_out):
    x_out[...] = x_in[...]
    e_out[...] = e_in[...]

    @pl.when(pl.program_id(0) == 0)
    def _once():
        y_out[...] = y_in[...]
        keep = _keep_2d()
        for r in range(_R):
            row = keep[r, :]
            sl = pl.ds(r * _C, _C)
            tr_out[sl] = jnp.logical_and(tr_in[sl], row)
            te_out[sl] = jnp.logical_and(te_in[sl], row)


def kernel(x, y, train_mask, test_mask, edge_index):
    once = lambda i: (0,)
    x_o, e_o, y_o, tr_o, te_o = pl.pallas_call(
        _body,
        grid=(_STEPS,),
        in_specs=[
            pl.BlockSpec((_XROWS, 128), lambda i: (i, 0)),
            pl.BlockSpec((_N,), once),
            pl.BlockSpec((_N,), once),
            pl.BlockSpec((_N,), once),
            pl.BlockSpec((2, _ECOLS), lambda i: (0, i)),
        ],
        out_specs=[
            pl.BlockSpec((_XROWS, 128), lambda i: (i, 0)),
            pl.BlockSpec((2, _ECOLS), lambda i: (0, i)),
            pl.BlockSpec((_N,), once),
            pl.BlockSpec((_N,), once),
            pl.BlockSpec((_N,), once),
        ],
        out_shape=(
            jax.ShapeDtypeStruct(x.shape, x.dtype),
            jax.ShapeDtypeStruct(edge_index.shape, edge_index.dtype),
            jax.ShapeDtypeStruct(y.shape, y.dtype),
            jax.ShapeDtypeStruct((_N,), jnp.bool_),
            jax.ShapeDtypeStruct((_N,), jnp.bool_),
        ),
    )(x, y, train_mask, test_mask, edge_index)
    return (x_o, e_o, y_o, tr_o, te_o)
